# baseline (speedup 1.0000x reference)
import jax
import jax.numpy as jnp
from jax.experimental import pallas as pl
from jax.experimental.pallas import tpu as pltpu

NUM_CLASSES_ = 1000
SPLIT_ = 896
N_ = 16384
R_ = 1024
NCHUNK_ = N_ // R_
K_ = 4


def _onehot_split(in_ref, out_ref, idxcol, buf, sems_a, sems_b):
    idxcol[...] = in_ref[...].reshape(N_, 1)
    cols = jax.lax.broadcasted_iota(jnp.int32, (R_, NUM_CLASSES_), 1)

    def copy_main(c, slot):
        return pltpu.make_async_copy(
            buf.at[slot, :, :SPLIT_],
            out_ref.at[pl.ds(c * R_, R_), pl.ds(0, SPLIT_)],
            sems_a.at[slot],
        )

    def copy_tail(c, slot):
        return pltpu.make_async_copy(
            buf.at[slot, :, SPLIT_:NUM_CLASSES_],
            out_ref.at[pl.ds(c * R_, R_), pl.ds(SPLIT_, NUM_CLASSES_ - SPLIT_)],
            sems_b.at[slot],
        )

    for c in range(NCHUNK_):
        slot = c % K_
        if c >= K_:
            copy_main(c - K_, slot).wait()
            copy_tail(c - K_, slot).wait()
        idx = idxcol[pl.ds(c * R_, R_), :]
        buf[slot] = (cols == idx).astype(buf.dtype)
        copy_main(c, slot).start()
        copy_tail(c, slot).start()

    for c in range(NCHUNK_ - K_, NCHUNK_):
        copy_main(c, c % K_).wait()
        copy_tail(c, c % K_).wait()


def kernel(input):
    return pl.pallas_call(
        _onehot_split,
        in_specs=[pl.BlockSpec(memory_space=pltpu.MemorySpace.VMEM)],
        out_specs=pl.BlockSpec(memory_space=pl.ANY),
        out_shape=jax.ShapeDtypeStruct((N_, NUM_CLASSES_), input.dtype),
        scratch_shapes=[
            pltpu.VMEM((N_, 1), jnp.int32),
            pltpu.VMEM((K_, R_, NUM_CLASSES_), jnp.int32),
            pltpu.SemaphoreType.DMA((K_,)),
            pltpu.SemaphoreType.DMA((K_,)),
        ],
    )(input)
